# baseline (speedup 1.0000x reference)
import functools

import jax
import jax.numpy as jnp
from jax import lax
from jax.experimental import pallas as pl
from jax.experimental.pallas import tpu as pltpu
from jax.experimental.pallas import tpu_sc as plsc

_ROWS = 32 * 8 * 16
_D = 128
_NW = 32
_RPW = _ROWS // _NW


def _sc_copy(k_hbm, v_hbm, ko_hbm, vo_hbm):
    wid = lax.axis_index("s") * 2 + lax.axis_index("c")
    base = wid * _RPW
    pltpu.sync_copy(k_hbm.at[pl.ds(base, _RPW)], ko_hbm.at[pl.ds(base, _RPW)])
    pltpu.sync_copy(v_hbm.at[pl.ds(base, _RPW)], vo_hbm.at[pl.ds(base, _RPW)])


def kernel(k_val, v_val, k_cache, v_cache):
    del k_cache, v_cache
    shape = k_val.shape
    k2 = k_val.reshape(_ROWS, _D)
    v2 = v_val.reshape(_ROWS, _D)
    mesh = plsc.VectorSubcoreMesh(core_axis_name="c", subcore_axis_name="s")
    ko, vo = pl.kernel(
        _sc_copy,
        mesh=mesh,
        out_type=(
            jax.ShapeDtypeStruct((_ROWS, _D), jnp.float32),
            jax.ShapeDtypeStruct((_ROWS, _D), jnp.float32),
        ),
    )(k2, v2)
    return (ko.reshape(shape), vo.reshape(shape))


# speedup vs baseline: 6.2506x; 6.2506x over previous
import functools

import jax
import jax.numpy as jnp
from jax import lax
from jax.experimental import pallas as pl
from jax.experimental.pallas import tpu as pltpu
from jax.experimental.pallas import tpu_sc as plsc

_ROWS = 32 * 8 * 16
_D = 128
_NW = 32
_RPW = _ROWS // _NW


def _sc_copy(k_hbm, v_hbm, ko_hbm, vo_hbm, kb, vb, sem_k, sem_v):
    wid = lax.axis_index("s") * 2 + lax.axis_index("c")
    sl = pl.ds(wid * _RPW, _RPW)
    ck = pltpu.async_copy(k_hbm.at[sl], kb, sem_k)
    cv = pltpu.async_copy(v_hbm.at[sl], vb, sem_v)
    ck.wait()
    cko = pltpu.async_copy(kb, ko_hbm.at[sl], sem_k)
    cv.wait()
    cvo = pltpu.async_copy(vb, vo_hbm.at[sl], sem_v)
    cko.wait()
    cvo.wait()


def kernel(k_val, v_val, k_cache, v_cache):
    del k_cache, v_cache
    shape = k_val.shape
    k2 = k_val.reshape(_ROWS, _D)
    v2 = v_val.reshape(_ROWS, _D)
    mesh = plsc.VectorSubcoreMesh(core_axis_name="c", subcore_axis_name="s")
    ko, vo = pl.kernel(
        _sc_copy,
        mesh=mesh,
        out_type=(
            jax.ShapeDtypeStruct((_ROWS, _D), jnp.float32),
            jax.ShapeDtypeStruct((_ROWS, _D), jnp.float32),
        ),
        scratch_types=[
            pltpu.VMEM((_RPW, _D), jnp.float32),
            pltpu.VMEM((_RPW, _D), jnp.float32),
            pltpu.SemaphoreType.DMA,
            pltpu.SemaphoreType.DMA,
        ],
    )(k2, v2)
    return (ko.reshape(shape), vo.reshape(shape))


# speedup vs baseline: 6.2835x; 1.0053x over previous
import functools

import jax
import jax.numpy as jnp
from jax import lax
from jax.experimental import pallas as pl
from jax.experimental.pallas import tpu as pltpu
from jax.experimental.pallas import tpu_sc as plsc

_ROWS = 32 * 8 * 16
_D = 128
_NW = 32
_RPW = _ROWS // _NW


_NCH = 4
_CR = _RPW // 2


def _sc_copy(k_hbm, v_hbm, ko_hbm, vo_hbm, kb, vb, sem_k, sem_v):
    wid = lax.axis_index("s") * 2 + lax.axis_index("c")
    base = wid * _RPW
    ins = []
    for t, (src, buf, sem) in enumerate(
        ((k_hbm, kb, sem_k), (v_hbm, vb, sem_v))):
        for c in range(2):
            sl = pl.ds(base + c * _CR, _CR)
            ins.append(pltpu.async_copy(src.at[sl], buf.at[pl.ds(c * _CR, _CR)], sem))
    outs = []
    for t, (dst, buf, sem) in enumerate(
        ((ko_hbm, kb, sem_k), (vo_hbm, vb, sem_v))):
        for c in range(2):
            sl = pl.ds(base + c * _CR, _CR)
            ins[t * 2 + c].wait()
            outs.append(pltpu.async_copy(buf.at[pl.ds(c * _CR, _CR)], dst.at[sl], sem))
    for o in outs:
        o.wait()


def kernel(k_val, v_val, k_cache, v_cache):
    del k_cache, v_cache
    shape = k_val.shape
    k2 = k_val.reshape(_ROWS, _D)
    v2 = v_val.reshape(_ROWS, _D)
    mesh = plsc.VectorSubcoreMesh(core_axis_name="c", subcore_axis_name="s")
    ko, vo = pl.kernel(
        _sc_copy,
        mesh=mesh,
        out_type=(
            jax.ShapeDtypeStruct((_ROWS, _D), jnp.float32),
            jax.ShapeDtypeStruct((_ROWS, _D), jnp.float32),
        ),
        scratch_types=[
            pltpu.VMEM((_RPW, _D), jnp.float32),
            pltpu.VMEM((_RPW, _D), jnp.float32),
            pltpu.SemaphoreType.DMA,
            pltpu.SemaphoreType.DMA,
        ],
    )(k2, v2)
    return (ko.reshape(shape), vo.reshape(shape))


# speedup vs baseline: 6.2958x; 1.0020x over previous
import functools

import jax
import jax.numpy as jnp
from jax import lax
from jax.experimental import pallas as pl
from jax.experimental.pallas import tpu as pltpu
from jax.experimental.pallas import tpu_sc as plsc

_ROWS = 32 * 8 * 16
_D = 128
_NW = 32
_RPW = _ROWS // _NW


def _sc_copy(k_hbm, v_hbm, ko_hbm, vo_hbm, kb, vb, sem_k, sem_v):
    wid = lax.axis_index("s") * 2 + lax.axis_index("c")
    sl = pl.ds(wid * _RPW, _RPW)
    ck = pltpu.async_copy(k_hbm.at[sl], kb, sem_k)
    cv = pltpu.async_copy(v_hbm.at[sl], vb, sem_v)
    ck.wait()
    cko = pltpu.async_copy(kb, ko_hbm.at[sl], sem_k)
    cv.wait()
    cvo = pltpu.async_copy(vb, vo_hbm.at[sl], sem_v)
    cko.wait()
    cvo.wait()


def kernel(k_val, v_val, k_cache, v_cache):
    del k_cache, v_cache
    shape = k_val.shape
    k2 = k_val.reshape(_ROWS, _D)
    v2 = v_val.reshape(_ROWS, _D)
    mesh = plsc.VectorSubcoreMesh(core_axis_name="c", subcore_axis_name="s")
    ko, vo = pl.kernel(
        _sc_copy,
        mesh=mesh,
        out_type=(
            jax.ShapeDtypeStruct((_ROWS, _D), jnp.float32),
            jax.ShapeDtypeStruct((_ROWS, _D), jnp.float32),
        ),
        scratch_types=[
            pltpu.VMEM((_RPW, _D), jnp.float32),
            pltpu.VMEM((_RPW, _D), jnp.float32),
            pltpu.SemaphoreType.DMA,
            pltpu.SemaphoreType.DMA,
        ],
    )(k2, v2)
    return (ko.reshape(shape), vo.reshape(shape))
